# Initial kernel scaffold; baseline (speedup 1.0000x reference)
import jax
import jax.numpy as jnp
from jax.experimental import pallas as pl


def kernel(x, edge_index, W1, b1, bn1_g, bn1_b, W2, b2, bn2_g, bn2_b, fc1_w, fc1_b, fc2_w, fc2_b, fc3_w, fc3_b, fc4_w, fc4_b, out_w, out_b):
    raise NotImplementedError("write your pallas kernel here")


# speedup vs baseline: 9.6780x; 9.6780x over previous
import functools

import jax
import jax.numpy as jnp
from jax import lax
from jax.experimental import pallas as pl
from jax.experimental.pallas import tpu as pltpu
from jax.experimental.pallas import tpu_sc as plsc

N = 10000
E = 320000
D = 128
H = 64
EPS = 1e-5

NC = 2
NS = 16
NW = NC * NS
EPT = E // NW
B = 128
NBT = 80
EPTP = NBT * B
NP = 10240
RPT = NP // NS

_mesh = plsc.VectorSubcoreMesh(core_axis_name="c", subcore_axis_name="s")


@functools.partial(
    pl.kernel,
    out_type=jax.ShapeDtypeStruct((NC * NP,), jnp.float32),
    mesh=_mesh,
    scratch_types=[
        pltpu.VMEM((NBT, B), jnp.int32),
        pltpu.VMEM((B,), jnp.float32),
        pltpu.VMEM((RPT,), jnp.float32),
        pltpu.VMEM_SHARED((NP,), jnp.float32),
    ],
)
def _deg_kernel(dst_hbm, out_hbm, di_v, ones_v, zbuf_v, acc_sh):
    c = lax.axis_index("c")
    s = lax.axis_index("s")
    wid = c * NS + s

    for i in range(B // 16):
        ones_v[pl.ds(i * 16, 16)] = jnp.ones((16,), jnp.float32)

    def _zero(i, _):
        zbuf_v[pl.ds(i * 16, 16)] = jnp.zeros((16,), jnp.float32)
        return ()

    lax.fori_loop(0, RPT // 16, _zero, ())

    pltpu.sync_copy(zbuf_v, acc_sh.at[pl.ds(s * RPT, RPT)])
    pltpu.sync_copy(dst_hbm.at[wid], di_v)
    plsc.subcore_barrier()

    def _body(j, _):
        pltpu.sync_copy(ones_v, acc_sh.at[di_v.at[j]], add=True)
        return ()

    lax.fori_loop(0, NBT, _body, ())
    plsc.subcore_barrier()

    pltpu.sync_copy(acc_sh.at[pl.ds(s * RPT, RPT)],
                    out_hbm.at[pl.ds(c * NP + s * RPT, RPT)])


SAC = NP - 8


def _vb(v, lane):
    return v.at[jnp.full((16,), lane, jnp.int32)].get(mode="promise_in_bounds")


@functools.partial(
    pl.kernel,
    out_type=jax.ShapeDtypeStruct((NC * NP, H), jnp.float32),
    mesh=_mesh,
    scratch_types=[
        pltpu.VMEM((NBT, B), jnp.int32),
        pltpu.VMEM((NBT, B), jnp.int32),
        pltpu.VMEM((B, H), jnp.float32),
        pltpu.VMEM_SHARED((NP, H), jnp.float32),
    ],
    compiler_params=pltpu.CompilerParams(use_tc_tiling_on_sc=False),
)
def _edge_kernel(hs_hbm, src_hbm, dst_hbm, zeros_hbm, out_hbm, si_v, di_v,
                 rows_v, acc_sh):
    c = lax.axis_index("c")
    s = lax.axis_index("s")
    wid = c * NS + s

    pltpu.sync_copy(zeros_hbm.at[pl.ds(s * RPT, RPT), :],
                    acc_sh.at[pl.ds(s * RPT, RPT), :])
    pltpu.sync_copy(src_hbm.at[wid], si_v)
    pltpu.sync_copy(dst_hbm.at[wid], di_v)
    plsc.subcore_barrier()

    def _body(j, _):
        pltpu.sync_copy(hs_hbm.at[si_v.at[j]], rows_v)
        pltpu.sync_copy(rows_v, acc_sh.at[di_v.at[j]], add=True)
        return ()

    lax.fori_loop(0, NBT, _body, ())
    plsc.subcore_barrier()

    pltpu.sync_copy(acc_sh.at[pl.ds(s * RPT, RPT), :],
                    out_hbm.at[pl.ds(c * NP + s * RPT, RPT), :])


_NACC = 16
_ROUNDS, _REM = divmod(N // 8, _NACC)
_INV_N = 1.0 / N


def _xla_rowmean(src_ref, f):
    def step(i, accs):
        return tuple(accs[k] + f(src_ref[pl.ds((i * _NACC + k) * 8, 8), :])
                     for k in range(_NACC))

    accs = lax.fori_loop(
        0, _ROUNDS, step,
        tuple(jnp.zeros((8, H), jnp.float32) for _ in range(_NACC)))
    accs = list(accs)
    for k in range(_REM):
        accs[k] = accs[k] + f(src_ref[pl.ds((_ROUNDS * _NACC + k) * 8, 8), :])
    acc = accs[0]
    for k in range(1, _NACC):
        acc = acc + accs[k]
    b = acc[0:4] + acc[4:8]
    c = b[0:2] + b[2:4]
    return (c[0:1] + c[1:2]) * _INV_N


def _tc1_body(deg_ref, x_ref, w1_ref, hs_ref, dinv_ref):
    d = deg_ref[pl.ds(0, N), :] + deg_ref[pl.ds(NP, N), :]
    dinv = lax.rsqrt(d + 1.0)
    h = jnp.dot(x_ref[...], w1_ref[...], preferred_element_type=jnp.float32)
    hs_ref[...] = h * dinv
    dinv_ref[...] = dinv


def _tc2_body(part_ref, hs_ref, dinv_ref, b_ref, g_ref, bb_ref, w2_ref,
              out_ref, t_ref):
    dinv = dinv_ref[...]
    p = part_ref[pl.ds(0, N), :] + part_ref[pl.ds(NP, N), :] + hs_ref[...]
    t = dinv * p + b_ref[...]
    t_ref[...] = jnp.maximum(t, 0.0)
    mu = _xla_rowmean(t_ref, lambda v: v)
    var = _xla_rowmean(t_ref, lambda v: (v - mu) ** 2)
    t = (t_ref[...] - mu) * lax.rsqrt(var + EPS) * g_ref[...] + bb_ref[...]
    h = jnp.dot(t, w2_ref[...], preferred_element_type=jnp.float32)
    out_ref[...] = h * dinv


def _tc3_body(part_ref, hs_ref, dinv_ref, b_ref, g_ref, bb_ref,
              fc1w_ref, fc1b_ref, fc2w_ref, fc2b_ref, fc3w_ref, fc3b_ref,
              fc4w_ref, fc4b_ref, ow_ref, ob_ref, out_ref, t_ref):
    dinv = dinv_ref[...]
    p = part_ref[pl.ds(0, N), :] + part_ref[pl.ds(NP, N), :] + hs_ref[...]
    t = dinv * p + b_ref[...]
    t_ref[...] = jnp.maximum(t, 0.0)
    mu = _xla_rowmean(t_ref, lambda v: v)
    var = _xla_rowmean(t_ref, lambda v: (v - mu) ** 2)
    t_ref[...] = ((t_ref[...] - mu) * lax.rsqrt(var + EPS) * g_ref[...]
                  + bb_ref[...])
    g = _xla_rowmean(t_ref, lambda v: v)
    g = jnp.maximum(jnp.dot(g, fc1w_ref[...],
                            preferred_element_type=jnp.float32)
                    + fc1b_ref[...], 0.0)
    g = jnp.maximum(jnp.dot(g, fc2w_ref[...],
                            preferred_element_type=jnp.float32)
                    + fc2b_ref[...], 0.0)
    g = jnp.maximum(jnp.dot(g, fc3w_ref[...],
                            preferred_element_type=jnp.float32)
                    + fc3b_ref[...], 0.0)
    g = jnp.maximum(jnp.dot(g, fc4w_ref[...],
                            preferred_element_type=jnp.float32)
                    + fc4b_ref[...], 0.0)
    out_ref[...] = (jnp.dot(g, ow_ref[...],
                            preferred_element_type=jnp.float32)
                    + ob_ref[...])


_tc1 = pl.pallas_call(
    _tc1_body,
    out_shape=[jax.ShapeDtypeStruct((N, H), jnp.float32),
               jax.ShapeDtypeStruct((N, 1), jnp.float32)],
)

_tc2 = pl.pallas_call(
    _tc2_body,
    out_shape=jax.ShapeDtypeStruct((N, H), jnp.float32),
    scratch_shapes=[pltpu.VMEM((N, H), jnp.float32)],
)

_tc3 = pl.pallas_call(
    _tc3_body,
    out_shape=jax.ShapeDtypeStruct((1, 2), jnp.float32),
    scratch_shapes=[pltpu.VMEM((N, H), jnp.float32)],
)


def kernel(x, edge_index, W1, b1, bn1_g, bn1_b, W2, b2, bn2_g, bn2_b,
           fc1_w, fc1_b, fc2_w, fc2_b, fc3_w, fc3_b, fc4_w, fc4_b,
           out_w, out_b):
    src_i = edge_index[0].astype(jnp.int32)
    dst_i = edge_index[1].astype(jnp.int32)
    dst_c = jnp.pad(dst_i.reshape(NW, EPT), ((0, 0), (0, EPTP - EPT)),
                    constant_values=N).reshape(NW, NBT, B)

    order = jnp.argsort(dst_i, stable=True)
    src_p = jnp.pad(src_i[order], (0, NW * EPTP - E),
                    constant_values=0).reshape(NW, NBT, B)
    dst_p = jnp.pad(dst_i[order], (0, NW * EPTP - E),
                    constant_values=N).reshape(NW, NBT, B)

    zeros2d = jnp.zeros((NP, H), jnp.float32)

    deg_p = _deg_kernel(dst_c).reshape(NC * NP, 1)

    hs1, dinv = _tc1(deg_p, x, W1)
    part1 = _edge_kernel(hs1, src_p, dst_p, zeros2d)
    hs2 = _tc2(part1, hs1, dinv, b1.reshape(1, H), bn1_g.reshape(1, H),
               bn1_b.reshape(1, H), W2)
    part2 = _edge_kernel(hs2, src_p, dst_p, zeros2d)
    out = _tc3(part2, hs2, dinv, b2.reshape(1, H), bn2_g.reshape(1, H),
               bn2_b.reshape(1, H),
               fc1_w, fc1_b.reshape(1, -1), fc2_w, fc2_b.reshape(1, -1),
               fc3_w, fc3_b.reshape(1, -1), fc4_w, fc4_b.reshape(1, -1),
               out_w, out_b.reshape(1, -1))
    return out
